# Initial kernel scaffold; baseline (speedup 1.0000x reference)
import jax
import jax.numpy as jnp
from jax.experimental import pallas as pl


def kernel(x, z, edge_attr, W0, b0, Wn1, bn1, Wn2, bn2, Wroot, bconv, W_ih, W_hh, b_ih, b_hh, Wl_ih, Wl_hh, bl_ih, bl_hh, W1, b1, W2, b2, edge_index, batch):
    raise NotImplementedError("write your pallas kernel here")


# speedup vs baseline: 1.3419x; 1.3419x over previous
import functools

import jax
import jax.numpy as jnp
from jax import lax
from jax.experimental import pallas as pl
from jax.experimental.pallas import tpu as pltpu
from jax.experimental.pallas import tpu_sc as plsc

N = 10000
E = 160000
INPUT_DIM = 128
DIM = 16
WIDTH = 64
B = 64
EDGE_DIM = 5

NC = 2
NS = 16
NW = NC * NS

EPW = 5120
E_PAD = NW * EPW
CHUNK = 128
NCHUNK = EPW // CHUNK
N_PAD = 10240
ROWS_PER_TILE = N_PAD // NS
DUMMY_ROW = N

TILE_E = 2048


def _node_mlp_body(feats_ref, w_ref, b_ref, out_ref):
    acc = jnp.dot(feats_ref[...], w_ref[...], preferred_element_type=jnp.float32)
    out_ref[...] = jnp.maximum(acc + b_ref[...], 0.0)


def _node_mlp(feats, w0, b0):
    return pl.pallas_call(
        _node_mlp_body,
        out_shape=jax.ShapeDtypeStruct((N, DIM), jnp.float32),
    )(feats, w0, b0.reshape(1, DIM))


def _we_body(ea_ref, wn1_ref, bn1_ref, wn2_ref, bn2_ref, we_ref):
    hidden = jnp.dot(ea_ref[...], wn1_ref[...], preferred_element_type=jnp.float32)
    hidden = jnp.maximum(hidden + bn1_ref[...], 0.0)
    we = jnp.dot(hidden, wn2_ref[...], preferred_element_type=jnp.float32)
    we_ref[...] = (we + bn2_ref[...]).astype(we_ref.dtype)


def _compute_we(ea_pad, wn1, bn1, wn2, bn2):
    grid = (E_PAD // TILE_E,)
    return pl.pallas_call(
        _we_body,
        grid=grid,
        in_specs=[
            pl.BlockSpec((TILE_E, EDGE_DIM), lambda i: (i, 0)),
            pl.BlockSpec((EDGE_DIM, WIDTH), lambda i: (0, 0)),
            pl.BlockSpec((1, WIDTH), lambda i: (0, 0)),
            pl.BlockSpec((WIDTH, DIM * DIM), lambda i: (0, 0)),
            pl.BlockSpec((1, DIM * DIM), lambda i: (0, 0)),
        ],
        out_specs=pl.BlockSpec((TILE_E, DIM * DIM), lambda i: (i, 0)),
        out_shape=jax.ShapeDtypeStruct((E_PAD, DIM * DIM), jnp.float32),
    )(ea_pad, wn1, bn1.reshape(1, WIDTH), wn2, bn2.reshape(1, DIM * DIM))


def _matvec_body(g_ref, we_ref, msg_ref):
    g = g_ref[...]
    we = we_ref[...].astype(jnp.float32)
    acc = g[:, 0:1] * we[:, 0:DIM]
    for d in range(1, DIM):
        acc = acc + g[:, d:d + 1] * we[:, DIM * d:DIM * (d + 1)]
    msg_ref[...] = acc


def _matvec(g, we):
    grid = (E_PAD // TILE_E,)
    return pl.pallas_call(
        _matvec_body,
        grid=grid,
        in_specs=[
            pl.BlockSpec((TILE_E, DIM), lambda i: (i, 0)),
            pl.BlockSpec((TILE_E, DIM * DIM), lambda i: (i, 0)),
        ],
        out_specs=pl.BlockSpec((TILE_E, DIM), lambda i: (i, 0)),
        out_shape=jax.ShapeDtypeStruct((E_PAD, DIM), jnp.float32),
    )(g, we)


def _gru_body(out_ref, h_ref, aggp_ref, cntp_ref, wroot_ref, bconv_ref,
              wihT_ref, whhT_ref, bih_ref, bhh_ref, hout_ref):
    out = out_ref[...]
    h = h_ref[...]
    aggp = aggp_ref[...]
    cntp = cntp_ref[...]
    cnt = jnp.maximum(cntp[0] + cntp[1], 1.0)
    agg = (aggp[0] + aggp[1]) / cnt
    m = jnp.dot(out, wroot_ref[...], preferred_element_type=jnp.float32)
    m = jnp.maximum(m + agg + bconv_ref[...], 0.0)
    gi = jnp.dot(m, wihT_ref[...], preferred_element_type=jnp.float32) + bih_ref[...]
    gh = jnp.dot(h, whhT_ref[...], preferred_element_type=jnp.float32) + bhh_ref[...]
    r = jax.nn.sigmoid(gi[:, :DIM] + gh[:, :DIM])
    u = jax.nn.sigmoid(gi[:, DIM:2 * DIM] + gh[:, DIM:2 * DIM])
    n = jnp.tanh(gi[:, 2 * DIM:] + r * gh[:, 2 * DIM:])
    hout_ref[...] = (1.0 - u) * n + u * h


def _gru(out, h, aggp, cntp, wroot, bconv, wihT, whhT, bih, bhh):
    return pl.pallas_call(
        _gru_body,
        out_shape=jax.ShapeDtypeStruct((N, DIM), jnp.float32),
    )(out, h, aggp, cntp, wroot, bconv.reshape(1, DIM),
      wihT, whhT, bih.reshape(1, 3 * DIM), bhh.reshape(1, 3 * DIM))


def _set2set_body(out_ref, batch_ref, wlihT_ref, wlhhT_ref, blih_ref, blhh_ref,
                  w1_ref, b1_ref, w2_ref, b2_ref, o2_ref):
    out = out_ref[...]
    batch = batch_ref[...]
    seg_iota = lax.broadcasted_iota(jnp.int32, (N, B), 1)
    m_hot = (batch == seg_iota).astype(jnp.float32)
    neg_inf = jnp.float32(-jnp.inf)

    q_star = jnp.zeros((B, 2 * DIM), jnp.float32)
    hs = jnp.zeros((B, DIM), jnp.float32)
    cs = jnp.zeros((B, DIM), jnp.float32)
    for _ in range(3):
        g = (jnp.dot(q_star, wlihT_ref[...], preferred_element_type=jnp.float32)
             + blih_ref[...]
             + jnp.dot(hs, wlhhT_ref[...], preferred_element_type=jnp.float32)
             + blhh_ref[...])
        i_g = jax.nn.sigmoid(g[:, :DIM])
        f_g = jax.nn.sigmoid(g[:, DIM:2 * DIM])
        g_g = jnp.tanh(g[:, 2 * DIM:3 * DIM])
        o_g = jax.nn.sigmoid(g[:, 3 * DIM:])
        cs = f_g * cs + i_g * g_g
        hs = o_g * jnp.tanh(cs)
        q = hs
        qn = jnp.dot(m_hot, q, preferred_element_type=jnp.float32,
                     precision=lax.Precision.HIGHEST)
        e = jnp.sum(out * qn, axis=1, keepdims=True)
        eb = jnp.where(m_hot > 0.0, e, neg_inf)
        emax_row = jnp.max(eb, axis=0, keepdims=True)
        emax_row = jnp.where(emax_row > neg_inf, emax_row, 0.0)
        emaxg = jnp.max(jnp.where(m_hot > 0.0, emax_row, neg_inf),
                        axis=1, keepdims=True)
        e2 = jnp.exp(e - emaxg)
        denom = lax.dot_general(m_hot, e2, (((0,), (0,)), ((), ())),
                                preferred_element_type=jnp.float32,
                                precision=lax.Precision.HIGHEST)
        ru = lax.dot_general(m_hot * e2, out, (((0,), (0,)), ((), ())),
                             preferred_element_type=jnp.float32,
                             precision=lax.Precision.HIGHEST)
        r_pool = ru / jnp.maximum(denom, 1e-16)
        q_star = jnp.concatenate([q, r_pool], axis=1)

    o1 = jnp.dot(q_star, w1_ref[...], preferred_element_type=jnp.float32)
    o1 = jnp.maximum(o1 + b1_ref[...], 0.0)
    o2 = jnp.dot(o1, w2_ref[...], preferred_element_type=jnp.float32)
    o2_ref[...] = o2 + b2_ref[...]


def _set2set(out, batch2d, wlihT, wlhhT, blih, blhh, w1, b1, w2, b2):
    return pl.pallas_call(
        _set2set_body,
        out_shape=jax.ShapeDtypeStruct((B, 1), jnp.float32),
    )(out, batch2d, wlihT, wlhhT, blih.reshape(1, 4 * DIM),
      blhh.reshape(1, 4 * DIM), w1, b1.reshape(1, DIM), w2, b2.reshape(1, 1))


def _make_sc_gather():
    mesh = plsc.VectorSubcoreMesh(core_axis_name="c", subcore_axis_name="s", num_cores=NC, num_subcores=NS)

    @functools.partial(
        pl.kernel,
        out_type=jax.ShapeDtypeStruct((E_PAD, DIM), jnp.float32),
        mesh=mesh,
        compiler_params=pltpu.CompilerParams(use_tc_tiling_on_sc=False),
        scratch_types=[
            pltpu.VMEM((NCHUNK, CHUNK), jnp.int32),
            pltpu.VMEM((EPW, DIM), jnp.float32),
            pltpu.SemaphoreType.DMA,
        ],
    )
    def gather_k(table_hbm, idx_hbm, g_hbm, idx_v, rows_v, sem):
        wid = lax.axis_index("s") * NC + lax.axis_index("c")
        pltpu.sync_copy(idx_hbm.at[pl.ds(wid * NCHUNK, NCHUNK)], idx_v)

        def fire(j, carry):
            pltpu.async_copy(table_hbm.at[idx_v.at[j]],
                             rows_v.at[pl.ds(j * CHUNK, CHUNK)], sem)
            return carry

        lax.fori_loop(0, NCHUNK, fire, 0)

        def drain(j, carry):
            pltpu.make_async_copy(table_hbm.at[pl.ds(0, CHUNK)],
                                  rows_v.at[pl.ds(0, CHUNK)], sem).wait()
            return carry

        lax.fori_loop(0, NCHUNK, drain, 0)
        pltpu.sync_copy(rows_v, g_hbm.at[pl.ds(wid * EPW, EPW)])

    return gather_k


def _make_sc_scatter(with_cnt):
    mesh = plsc.VectorSubcoreMesh(core_axis_name="c", subcore_axis_name="s", num_cores=NC, num_subcores=NS)
    out_type = [jax.ShapeDtypeStruct((NC, N_PAD, DIM), jnp.float32)]
    scratch = [
        pltpu.VMEM((NCHUNK, CHUNK), jnp.int32),
        pltpu.VMEM((EPW, DIM), jnp.float32),
        pltpu.VMEM((ROWS_PER_TILE, DIM), jnp.float32),
        pltpu.VMEM_SHARED((N_PAD, DIM), jnp.float32),
    ]
    if with_cnt:
        out_type.append(jax.ShapeDtypeStruct((NC, N_PAD, DIM), jnp.float32))
        scratch.append(pltpu.VMEM((CHUNK, DIM), jnp.float32))
        scratch.append(pltpu.VMEM_SHARED((N_PAD, DIM), jnp.float32))

    @functools.partial(
        pl.kernel,
        out_type=tuple(out_type),
        mesh=mesh,
        compiler_params=pltpu.CompilerParams(use_tc_tiling_on_sc=False),
        scratch_types=scratch,
    )
    def scatter_k(msg_hbm, idx_hbm, *refs):
        if with_cnt:
            (agg_hbm, cnt_hbm, idx_v, rows_v, zbuf_v, sh_agg,
             obuf_v, sh_cnt) = refs
        else:
            agg_hbm, idx_v, rows_v, zbuf_v, sh_agg = refs
        cid = lax.axis_index("c")
        sid = lax.axis_index("s")
        wid = sid * NC + cid

        def zrow(i, carry):
            zbuf_v[i] = jnp.zeros((DIM,), jnp.float32)
            return carry

        lax.fori_loop(0, ROWS_PER_TILE, zrow, 0)
        pltpu.sync_copy(zbuf_v, sh_agg.at[pl.ds(sid * ROWS_PER_TILE,
                                                ROWS_PER_TILE)])
        if with_cnt:
            def orow(i, carry):
                obuf_v[i] = jnp.ones((DIM,), jnp.float32)
                return carry

            lax.fori_loop(0, CHUNK, orow, 0)
            pltpu.sync_copy(zbuf_v, sh_cnt.at[pl.ds(sid * ROWS_PER_TILE,
                                                    ROWS_PER_TILE)])
        plsc.subcore_barrier()

        pltpu.sync_copy(idx_hbm.at[pl.ds(wid * NCHUNK, NCHUNK)], idx_v)
        pltpu.sync_copy(msg_hbm.at[pl.ds(wid * EPW, EPW)], rows_v)

        def step(j, carry):
            pltpu.sync_copy(rows_v.at[pl.ds(j * CHUNK, CHUNK)],
                            sh_agg.at[idx_v.at[j]], add=True)
            if with_cnt:
                pltpu.sync_copy(obuf_v, sh_cnt.at[idx_v.at[j]], add=True)
            return carry

        lax.fori_loop(0, NCHUNK, step, 0)
        plsc.subcore_barrier()

        row0 = sid * ROWS_PER_TILE
        pltpu.sync_copy(sh_agg.at[pl.ds(row0, ROWS_PER_TILE)],
                        agg_hbm.at[cid].at[pl.ds(row0, ROWS_PER_TILE)])
        if with_cnt:
            pltpu.sync_copy(sh_cnt.at[pl.ds(row0, ROWS_PER_TILE)],
                            cnt_hbm.at[cid].at[pl.ds(row0, ROWS_PER_TILE)])

    return scatter_k


_sc_cache = {}


def _sc_gather(table, src2d):
    fn = _sc_cache.get("gather")
    if fn is None:
        fn = _sc_cache["gather"] = _make_sc_gather()
    return fn(table, src2d)


def _sc_scatter(msg, dst2d, with_cnt):
    fn = _sc_cache.get(("scatter", with_cnt))
    if fn is None:
        fn = _sc_cache[("scatter", with_cnt)] = _make_sc_scatter(with_cnt)
    if with_cnt:
        return fn(msg, dst2d)
    return fn(msg, dst2d)[0], None


def kernel(x, z, edge_attr, W0, b0, Wn1, bn1, Wn2, bn2, Wroot, bconv,
           W_ih, W_hh, b_ih, b_hh, Wl_ih, Wl_hh, bl_ih, bl_hh, W1, b1, W2, b2,
           edge_index, batch):
    feats = jnp.concatenate([x, z[:, None]], axis=1)
    src = edge_index[0]
    dst = edge_index[1]
    pad = E_PAD - E
    src2d = jnp.concatenate(
        [src, jnp.zeros((pad,), jnp.int32)]).reshape(E_PAD // CHUNK, CHUNK)
    dst2d = jnp.concatenate(
        [dst, jnp.full((pad,), DUMMY_ROW, jnp.int32)]).reshape(
            E_PAD // CHUNK, CHUNK)
    ea_pad = jnp.concatenate(
        [edge_attr, jnp.zeros((pad, EDGE_DIM), jnp.float32)], axis=0)

    out = _node_mlp(feats, W0, b0)
    h = out
    we = _compute_we(ea_pad, Wn1, bn1, Wn2, bn2)

    wihT = W_ih.T
    whhT = W_hh.T
    cntp = None
    for it in range(3):
        g = _sc_gather(out, src2d)
        msg = _matvec(g, we)
        aggp, cnt_new = _sc_scatter(msg, dst2d, it == 0)
        if it == 0:
            cntp = cnt_new
        h = _gru(out, h, aggp[:, :N, :], cntp[:, :N, :],
                 Wroot, bconv, wihT, whhT, b_ih, b_hh)
        out = h

    batch2d = batch.reshape(N, 1)
    o2 = _set2set(out, batch2d, Wl_ih.T, Wl_hh.T, bl_ih, bl_hh,
                  W1, b1, W2, b2)
    return o2.reshape(-1)
